# baseline (speedup 1.0000x reference)
import jax
import jax.numpy as jnp
from jax.experimental import pallas as pl

H = 128


def _gru_body(x_ref, w_ref, b_ref, o_ref):
    x = x_ref[...]
    s = jnp.sum(x * x, axis=1, keepdims=True)
    h = x * jax.lax.rsqrt(jnp.maximum(s, 1e-24))
    g = jnp.dot(h, w_ref[...], preferred_element_type=jnp.float32) + b_ref[...]
    r = 0.5 * (1.0 + jnp.tanh(g[:, 0:H]))
    z = 0.5 * (1.0 + jnp.tanh(g[:, H:2 * H]))
    c = jnp.tanh(g[:, 2 * H:3 * H] + r * g[:, 3 * H:4 * H])
    o_ref[...] = c + z * (h - c)


def kernel(dynamic_emb, emb_rel, W_ih_rel, W_hh_rel, b_ih_rel, b_hh_rel,
           W_ih_ent, W_hh_ent, b_ih_ent, b_hh_ent, r_to_e, seg_ids):
    N, Hd = dynamic_emb.shape
    W_rz = 0.5 * (W_ih_ent[0:2 * H] + W_hh_ent[0:2 * H]).T
    W_in = W_ih_ent[2 * H:3 * H].T
    W_hn = W_hh_ent[2 * H:3 * H].T
    W = jnp.concatenate([W_rz, W_in, W_hn], axis=1)
    b = jnp.concatenate([
        0.5 * (b_ih_ent[0:2 * H] + b_hh_ent[0:2 * H]),
        b_ih_ent[2 * H:3 * H],
        b_hh_ent[2 * H:3 * H]], axis=0)[None, :]

    B = 1000
    out = pl.pallas_call(
        _gru_body,
        grid=(N // B,),
        in_specs=[
            pl.BlockSpec((B, Hd), lambda i: (i, 0)),
            pl.BlockSpec((Hd, 4 * H), lambda i: (0, 0)),
            pl.BlockSpec((1, 4 * H), lambda i: (0, 0)),
        ],
        out_specs=pl.BlockSpec((B, Hd), lambda i: (i, 0)),
        out_shape=jax.ShapeDtypeStruct((N, Hd), jnp.float32),
    )(dynamic_emb, W, b)
    return out


# speedup vs baseline: 1.2881x; 1.2881x over previous
import jax
import jax.numpy as jnp
from jax.experimental import pallas as pl

H = 128


def _gru_body(x_ref, wih_ref, whh_ref, bih_ref, bhh_ref, o_ref):
    x = x_ref[...]
    s = jnp.sum(x * x, axis=1, keepdims=True)
    h = x * jax.lax.rsqrt(jnp.maximum(s, 1e-24))
    wih = wih_ref[...]
    whh = whh_ref[...]
    bih = bih_ref[...]
    bhh = bhh_ref[...]
    w_rz = 0.5 * (wih[0:2 * H] + whh[0:2 * H])
    dn = (((1,), (1,)), ((), ()))
    g_rz = jax.lax.dot_general(h, w_rz, dn, preferred_element_type=jnp.float32)
    g_rz = g_rz + 0.5 * (bih[:, 0:2 * H] + bhh[:, 0:2 * H])
    g_in = jax.lax.dot_general(h, wih[2 * H:], dn,
                               preferred_element_type=jnp.float32) + bih[:, 2 * H:]
    g_hn = jax.lax.dot_general(h, whh[2 * H:], dn,
                               preferred_element_type=jnp.float32) + bhh[:, 2 * H:]
    r = 0.5 * (1.0 + jnp.tanh(g_rz[:, 0:H]))
    z = 0.5 * (1.0 + jnp.tanh(g_rz[:, H:2 * H]))
    c = jnp.tanh(g_in + r * g_hn)
    o_ref[...] = c + z * (h - c)


def kernel(dynamic_emb, emb_rel, W_ih_rel, W_hh_rel, b_ih_rel, b_hh_rel,
           W_ih_ent, W_hh_ent, b_ih_ent, b_hh_ent, r_to_e, seg_ids):
    N, Hd = dynamic_emb.shape
    B = 1000
    out = pl.pallas_call(
        _gru_body,
        grid=(N // B,),
        in_specs=[
            pl.BlockSpec((B, Hd), lambda i: (i, 0)),
            pl.BlockSpec((3 * H, Hd), lambda i: (0, 0)),
            pl.BlockSpec((3 * H, Hd), lambda i: (0, 0)),
            pl.BlockSpec((1, 3 * H), lambda i: (0, 0)),
            pl.BlockSpec((1, 3 * H), lambda i: (0, 0)),
        ],
        out_specs=pl.BlockSpec((B, Hd), lambda i: (i, 0)),
        out_shape=jax.ShapeDtypeStruct((N, Hd), jnp.float32),
    )(dynamic_emb, W_ih_ent, W_hh_ent, b_ih_ent[None, :], b_hh_ent[None, :])
    return out


# speedup vs baseline: 1.6080x; 1.2483x over previous
import jax
import jax.numpy as jnp
from jax.experimental import pallas as pl

H = 128


def _gru_body(x_ref, wih_ref, whh_ref, bih_ref, bhh_ref, o_ref):
    x = x_ref[...]
    s = jnp.sum(x * x, axis=1, keepdims=True)
    h = x * jax.lax.rsqrt(jnp.maximum(s, 1e-24))
    wih = wih_ref[...]
    whh = whh_ref[...]
    bih = bih_ref[...]
    bhh = bhh_ref[...]
    w_rz = 0.5 * (wih[0:2 * H] + whh[0:2 * H])
    dn = (((1,), (1,)), ((), ()))
    g_rz = jax.lax.dot_general(h, w_rz, dn, preferred_element_type=jnp.float32)
    g_rz = g_rz + 0.5 * (bih[:, 0:2 * H] + bhh[:, 0:2 * H])
    g_in = jax.lax.dot_general(h, wih[2 * H:], dn,
                               preferred_element_type=jnp.float32) + bih[:, 2 * H:]
    g_hn = jax.lax.dot_general(h, whh[2 * H:], dn,
                               preferred_element_type=jnp.float32) + bhh[:, 2 * H:]
    r = 0.5 * (1.0 + jnp.tanh(g_rz[:, 0:H]))
    z = 0.5 * (1.0 + jnp.tanh(g_rz[:, H:2 * H]))
    c = jnp.tanh(g_in + r * g_hn)
    o_ref[...] = c + z * (h - c)


def kernel(dynamic_emb, emb_rel, W_ih_rel, W_hh_rel, b_ih_rel, b_hh_rel,
           W_ih_ent, W_hh_ent, b_ih_ent, b_hh_ent, r_to_e, seg_ids):
    N, Hd = dynamic_emb.shape
    B = 2000
    out = pl.pallas_call(
        _gru_body,
        grid=(N // B,),
        in_specs=[
            pl.BlockSpec((B, Hd), lambda i: (i, 0)),
            pl.BlockSpec((3 * H, Hd), lambda i: (0, 0)),
            pl.BlockSpec((3 * H, Hd), lambda i: (0, 0)),
            pl.BlockSpec((1, 3 * H), lambda i: (0, 0)),
            pl.BlockSpec((1, 3 * H), lambda i: (0, 0)),
        ],
        out_specs=pl.BlockSpec((B, Hd), lambda i: (i, 0)),
        out_shape=jax.ShapeDtypeStruct((N, Hd), jnp.float32),
    )(dynamic_emb, W_ih_ent, W_hh_ent, b_ih_ent[None, :], b_hh_ent[None, :])
    return out


# speedup vs baseline: 1.6116x; 1.0022x over previous
import jax
import jax.numpy as jnp
from jax.experimental import pallas as pl

H = 128


def _gru_body(x_ref, wih_ref, whh_ref, bih_ref, bhh_ref, o_ref):
    x = x_ref[...]
    s = jnp.sum(x * x, axis=1, keepdims=True)
    h = x * jax.lax.rsqrt(jnp.maximum(s, 1e-24))
    wih = wih_ref[...]
    whh = whh_ref[...]
    bih = bih_ref[...]
    bhh = bhh_ref[...]
    w_rz = 0.5 * (wih[0:2 * H] + whh[0:2 * H])
    dn = (((1,), (1,)), ((), ()))
    g_rz = jax.lax.dot_general(h, w_rz, dn, preferred_element_type=jnp.float32)
    g_rz = g_rz + 0.5 * (bih[:, 0:2 * H] + bhh[:, 0:2 * H])
    g_in = jax.lax.dot_general(h, wih[2 * H:], dn,
                               preferred_element_type=jnp.float32) + bih[:, 2 * H:]
    g_hn = jax.lax.dot_general(h, whh[2 * H:], dn,
                               preferred_element_type=jnp.float32) + bhh[:, 2 * H:]
    r = 0.5 * (1.0 + jnp.tanh(g_rz[:, 0:H]))
    z = 0.5 * (1.0 + jnp.tanh(g_rz[:, H:2 * H]))
    c = jnp.tanh(g_in + r * g_hn)
    o_ref[...] = c + z * (h - c)


def kernel(dynamic_emb, emb_rel, W_ih_rel, W_hh_rel, b_ih_rel, b_hh_rel,
           W_ih_ent, W_hh_ent, b_ih_ent, b_hh_ent, r_to_e, seg_ids):
    N, Hd = dynamic_emb.shape
    B = 5000
    out = pl.pallas_call(
        _gru_body,
        grid=(N // B,),
        in_specs=[
            pl.BlockSpec((B, Hd), lambda i: (i, 0)),
            pl.BlockSpec((3 * H, Hd), lambda i: (0, 0)),
            pl.BlockSpec((3 * H, Hd), lambda i: (0, 0)),
            pl.BlockSpec((1, 3 * H), lambda i: (0, 0)),
            pl.BlockSpec((1, 3 * H), lambda i: (0, 0)),
        ],
        out_specs=pl.BlockSpec((B, Hd), lambda i: (i, 0)),
        out_shape=jax.ShapeDtypeStruct((N, Hd), jnp.float32),
    )(dynamic_emb, W_ih_ent, W_hh_ent, b_ih_ent[None, :], b_hh_ent[None, :])
    return out
